# baseline (speedup 1.0000x reference)
import functools

import jax
import jax.numpy as jnp
from jax import lax
from jax.experimental import pallas as pl
from jax.experimental.pallas import tpu as pltpu
from jax.experimental.pallas import tpu_sc as plsc

N = 10000
E = 160000
DIM = 256
HID = 512
NCLS = 2

NC = 2
NS = 16
NPAD = 10240
FQ = 64
P16 = 16
NPT = NPAD // NS

EPT_A = E // NS
CH_A = 400
NCH_A = EPT_A // CH_A
NBUF = 2


CB = 8


def _make_sc_layer1(p):
    def body(xflat, srcq, ei4, z64, out,
             srcgb, dstb, rows0, rows1, acc, sem0, sem1):
        c = lax.axis_index("c")
        s = lax.axis_index("s")
        bufs = (rows0, rows1)
        sems = (sem0, sem1)

        pltpu.sync_copy(ei4.at[1, s], dstb)
        pltpu.sync_copy(z64, acc.at[pl.ds(s * NPT, NPT)])
        plsc.subcore_barrier()

        q = 2 * p + c
        pltpu.sync_copy(srcq.at[pl.ds(q * E + s * EPT_A, EPT_A)], srcgb)

        hg = {}
        for i in range(NBUF):
            hg[i] = pltpu.async_copy(
                xflat.at[srcgb.at[pl.ds(i * CH_A, CH_A)]], bufs[i], sems[i])
        for i in range(NCH_A):
            buf = bufs[i % NBUF]
            hg[i].wait()
            pltpu.sync_copy(buf, acc.at[dstb.at[i]], add=True)
            if i + NBUF < NCH_A:
                hg[i + NBUF] = pltpu.async_copy(
                    xflat.at[srcgb.at[pl.ds((i + NBUF) * CH_A, CH_A)]],
                    buf, sems[i % NBUF])

        plsc.subcore_barrier()
        pltpu.sync_copy(acc.at[pl.ds(s * NPT, NPT)],
                        out.at[c, pl.ds(s * NPT, NPT)])

    return pl.kernel(
        body,
        mesh=plsc.VectorSubcoreMesh(core_axis_name="c", subcore_axis_name="s"),
        out_type=jax.ShapeDtypeStruct((2, NPAD, FQ), jnp.float32),
        scratch_types=[
            pltpu.VMEM((EPT_A,), jnp.int32),
            pltpu.VMEM((NCH_A, CH_A), jnp.int32),
            pltpu.VMEM((CH_A, FQ), jnp.float32),
            pltpu.VMEM((CH_A, FQ), jnp.float32),
            pltpu.VMEM_SHARED((NPAD, FQ), jnp.float32),
            pltpu.SemaphoreType.DMA,
            pltpu.SemaphoreType.DMA,
        ],
        compiler_params=pltpu.CompilerParams(use_tc_tiling_on_sc=False),
    )


_sc_layer1_p0 = _make_sc_layer1(0)
_sc_layer1_p1 = _make_sc_layer1(1)


def _sc_count(ei4, ones8, z8, cnt_out, dstb, onesb, cacc):
    c = lax.axis_index("c")
    s = lax.axis_index("s")
    pltpu.sync_copy(ei4.at[1, s], dstb)
    pltpu.sync_copy(ones8, onesb)
    pltpu.sync_copy(z8, cacc.at[pl.ds(s * NPT, NPT)])
    plsc.subcore_barrier()
    for i in range(NCH_A):
        @pl.when(c == (i % 2))
        def _():
            pltpu.sync_copy(onesb, cacc.at[dstb.at[i]], add=True)
    plsc.subcore_barrier()
    pltpu.sync_copy(cacc.at[pl.ds(s * NPT, NPT)],
                    cnt_out.at[c, pl.ds(s * NPT, NPT)])


_sc_count_call = functools.partial(
    pl.kernel,
    mesh=plsc.VectorSubcoreMesh(core_axis_name="c", subcore_axis_name="s"),
    out_type=jax.ShapeDtypeStruct((2, NPAD, CB), jnp.float32),
    scratch_types=[
        pltpu.VMEM((NCH_A, CH_A), jnp.int32),
        pltpu.VMEM((CH_A, CB), jnp.float32),
        pltpu.VMEM_SHARED((NPAD, CB), jnp.float32),
    ],
    compiler_params=pltpu.CompilerParams(use_tc_tiling_on_sc=False),
)(_sc_count)

EPT_B = E // NS
CH_B = 1000
NCH_B = EPT_B // CH_B


def _sc_layer2(p16, ei4, z16, out, srcb, dstb, rowsA, rowsB,
               abuf, pbuf, obuf, acc, semA, semB):
    c = lax.axis_index("c")
    s = lax.axis_index("s")

    @pl.when(c == 0)
    def _():
        pltpu.sync_copy(ei4.at[0, s], srcb)
        pltpu.sync_copy(ei4.at[1, s], dstb)
        pltpu.sync_copy(z16, acc.at[pl.ds(s * NPT, NPT)])
        plsc.subcore_barrier()

        bufs = (rowsA, rowsB)
        sems = (semA, semB)
        hg = {}
        for i in range(2):
            hg[i] = pltpu.async_copy(p16.at[srcb.at[i]], bufs[i], sems[i])
        for i in range(NCH_B):
            buf = bufs[i % 2]
            hg[i].wait()
            pltpu.sync_copy(buf, acc.at[dstb.at[i]], add=True)
            if i + 2 < NCH_B:
                hg[i + 2] = pltpu.async_copy(p16.at[srcb.at[i + 2]], buf,
                                             sems[i % 2])

        plsc.subcore_barrier()
        pltpu.sync_copy(acc.at[pl.ds(s * NPT, NPT)], abuf)
        pltpu.sync_copy(p16.at[pl.ds(s * NPT, NPT)], pbuf)

        def body(r, _):
            av = abuf[r]
            pv = pbuf[r]
            inv = pv[4]
            o0 = av[0] * inv + pv[NCLS]
            o1 = av[1] * inv + pv[NCLS + 1]
            lane = lax.iota(jnp.int32, 16)
            obuf[r] = jnp.where(lane == 0, o0, jnp.where(lane == 1, o1, 0.0))
            return 0

        lax.fori_loop(0, NPT, body, 0)
        pltpu.sync_copy(obuf, out.at[pl.ds(s * NPT, NPT)])


_sc_layer2_call = functools.partial(
    pl.kernel,
    mesh=plsc.VectorSubcoreMesh(core_axis_name="c", subcore_axis_name="s"),
    out_type=jax.ShapeDtypeStruct((NPAD, P16), jnp.float32),
    scratch_types=[
        pltpu.VMEM((NCH_B, CH_B), jnp.int32),
        pltpu.VMEM((NCH_B, CH_B), jnp.int32),
        pltpu.VMEM((CH_B, P16), jnp.float32),
        pltpu.VMEM((CH_B, P16), jnp.float32),
        pltpu.VMEM((NPT, P16), jnp.float32),
        pltpu.VMEM((NPT, P16), jnp.float32),
        pltpu.VMEM((NPT, P16), jnp.float32),
        pltpu.VMEM_SHARED((NPAD, P16), jnp.float32),
        pltpu.SemaphoreType.DMA,
        pltpu.SemaphoreType.DMA,
    ],
    compiler_params=pltpu.CompilerParams(use_tc_tiling_on_sc=False),
)(_sc_layer2)

BM = 2048


def _tc_root(x, b, b1, xb_out):
    xb_out[...] = (jnp.dot(x[...], b[...], preferred_element_type=jnp.float32)
                   + b1[...]).astype(jnp.bfloat16)


def _tc_root_call(x, b, b1):
    return pl.pallas_call(
        _tc_root,
        grid=(NPAD // BM,),
        in_specs=[
            pl.BlockSpec((BM, DIM), lambda i: (i, 0)),
            pl.BlockSpec((DIM, HID), lambda i: (0, 0)),
            pl.BlockSpec((1, HID), lambda i: (0, 0)),
        ],
        out_specs=pl.BlockSpec((BM, HID), lambda i: (i, 0)),
        out_shape=jax.ShapeDtypeStruct((NPAD, HID), jnp.bfloat16),
    )(x, b, b1)


def _tc_hidden(s0, s1, s2, s3, c0, c1, xb, a, w2l, w2r, b2, p16_out):
    cnt = c0[0][:, 0:1] + c1[0][:, 0:1]
    inv = 1.0 / jnp.maximum(cnt, 1.0)
    sfull = jnp.concatenate([s0[0], s1[0], s2[0], s3[0]], axis=1)
    zs = jnp.dot(sfull, a[...], preferred_element_type=jnp.float32)
    h = jnp.maximum(zs * inv + xb[...].astype(jnp.float32), 0.0)
    pl_ = jnp.dot(h, w2l[...], preferred_element_type=jnp.float32)
    pr = jnp.dot(h, w2r[...], preferred_element_type=jnp.float32) + b2[...]
    col = lax.broadcasted_iota(jnp.int32, (BM, P16), 1)
    inv_col = jnp.where(col == 4, inv, 0.0)
    zpad = jnp.zeros((BM, P16 - 2 * NCLS), jnp.float32)
    p16_out[...] = (jnp.concatenate([pl_, pr, zpad], axis=1) + inv_col)


def _quarter_spec(q):
    return pl.BlockSpec((1, BM, FQ), lambda i, _q=q: (_q, i, 0))


def _tc_hidden_call(s01b, s23b, cnt, xb, a, w2l, w2r, b2):
    return pl.pallas_call(
        _tc_hidden,
        grid=(NPAD // BM,),
        in_specs=[
            _quarter_spec(0), _quarter_spec(1), _quarter_spec(0),
            _quarter_spec(1),
            pl.BlockSpec((1, BM, CB), lambda i: (0, i, 0)),
            pl.BlockSpec((1, BM, CB), lambda i: (1, i, 0)),
            pl.BlockSpec((BM, HID), lambda i: (i, 0)),
            pl.BlockSpec((DIM, HID), lambda i: (0, 0)),
            pl.BlockSpec((HID, NCLS), lambda i: (0, 0)),
            pl.BlockSpec((HID, NCLS), lambda i: (0, 0)),
            pl.BlockSpec((1, NCLS), lambda i: (0, 0)),
        ],
        out_specs=pl.BlockSpec((BM, P16), lambda i: (i, 0)),
        out_shape=jax.ShapeDtypeStruct((NPAD, P16), jnp.float32),
    )(s01b, s01b, s23b, s23b, cnt, cnt, xb, a, w2l, w2r, b2)


def kernel(x, edge_index, W1l, b1l, W1r, W2l, b2l, W2r):
    src = edge_index[0].astype(jnp.int32)
    dst = edge_index[1].astype(jnp.int32)

    xflat = x.reshape(4 * N, FQ)
    srcq = (src[None, :] * 4 + jnp.arange(4, dtype=jnp.int32)[:, None]).ravel()
    ei = jnp.stack([src, dst]) if edge_index.dtype != jnp.int32 else edge_index
    ei4a = ei.reshape(2, NS, NCH_A, CH_A)
    ones8 = jnp.ones((CH_A, CB), jnp.float32)
    z64 = jnp.zeros((NPT, FQ), jnp.float32)
    z8 = jnp.zeros((NPT, CB), jnp.float32)
    z16 = jnp.zeros((NPT, P16), jnp.float32)

    s01 = _sc_layer1_p0(xflat, srcq, ei4a, z64)
    s23 = _sc_layer1_p1(xflat, srcq, ei4a, z64)
    cnt = _sc_count_call(ei4a, ones8, z8)
    xb = _tc_root_call(x.astype(jnp.bfloat16), W1r.T.astype(jnp.bfloat16),
                       b1l.reshape(1, HID))

    p16 = _tc_hidden_call(s01.astype(jnp.bfloat16), s23.astype(jnp.bfloat16),
                          cnt, xb,
                          W1l.T.astype(jnp.bfloat16), W2l.T, W2r.T,
                          b2l.reshape(1, NCLS))

    ei4b = ei.reshape(2, NS, NCH_B, CH_B)
    out16 = _sc_layer2_call(p16, ei4b, z16)

    return out16[:N, :NCLS]


# speedup vs baseline: 1.0302x; 1.0302x over previous
import functools

import jax
import jax.numpy as jnp
from jax import lax
from jax.experimental import pallas as pl
from jax.experimental.pallas import tpu as pltpu
from jax.experimental.pallas import tpu_sc as plsc

N = 10000
E = 160000
DIM = 256
HID = 512
NCLS = 2

NC = 2
NS = 16
NPAD = 10240
FQ = 64
P16 = 16
NPT = NPAD // NS

EPT_A = E // NS
CH_A = 400
NCH_A = EPT_A // CH_A
NBUF = 2


CB = 8


def _make_sc_layer1(p):
    def body(xflat, srcq, ei4, ones8, z64, z8, *refs):
        if p == 0:
            out, cnt_out, srcgb, dstb, rows0, rows1, onesb, acc, cacc, \
                sem0, sem1 = refs
        else:
            out, srcgb, dstb, rows0, rows1, acc, sem0, sem1 = refs
        c = lax.axis_index("c")
        s = lax.axis_index("s")
        bufs = (rows0, rows1)
        sems = (sem0, sem1)

        pltpu.sync_copy(ei4.at[1, s], dstb)
        if p == 0:
            pltpu.sync_copy(ones8, onesb)
            pltpu.sync_copy(z8, cacc.at[pl.ds(s * NPT, NPT)])
        pltpu.sync_copy(z64, acc.at[pl.ds(s * NPT, NPT)])
        plsc.subcore_barrier()

        q = 2 * p + c
        pltpu.sync_copy(srcq.at[pl.ds(q * E + s * EPT_A, EPT_A)], srcgb)

        hg = {}
        for i in range(NBUF):
            hg[i] = pltpu.async_copy(
                xflat.at[srcgb.at[pl.ds(i * CH_A, CH_A)]], bufs[i], sems[i])
        for i in range(NCH_A):
            buf = bufs[i % NBUF]
            hg[i].wait()
            pltpu.sync_copy(buf, acc.at[dstb.at[i]], add=True)
            if p == 0:
                @pl.when(c == (i % 2))
                def _():
                    pltpu.sync_copy(onesb, cacc.at[dstb.at[i]], add=True)
            if i + NBUF < NCH_A:
                hg[i + NBUF] = pltpu.async_copy(
                    xflat.at[srcgb.at[pl.ds((i + NBUF) * CH_A, CH_A)]],
                    buf, sems[i % NBUF])

        plsc.subcore_barrier()
        pltpu.sync_copy(acc.at[pl.ds(s * NPT, NPT)],
                        out.at[c, pl.ds(s * NPT, NPT)])
        if p == 0:
            pltpu.sync_copy(cacc.at[pl.ds(s * NPT, NPT)],
                            cnt_out.at[c, pl.ds(s * NPT, NPT)])

    out_type = [jax.ShapeDtypeStruct((2, NPAD, FQ), jnp.float32)]
    scratch = [
        pltpu.VMEM((EPT_A,), jnp.int32),
        pltpu.VMEM((NCH_A, CH_A), jnp.int32),
        pltpu.VMEM((CH_A, FQ), jnp.float32),
        pltpu.VMEM((CH_A, FQ), jnp.float32),
    ]
    if p == 0:
        out_type.append(jax.ShapeDtypeStruct((2, NPAD, CB), jnp.float32))
        scratch.append(pltpu.VMEM((CH_A, CB), jnp.float32))
    scratch.append(pltpu.VMEM_SHARED((NPAD, FQ), jnp.float32))
    if p == 0:
        scratch.append(pltpu.VMEM_SHARED((NPAD, CB), jnp.float32))
    scratch += [pltpu.SemaphoreType.DMA, pltpu.SemaphoreType.DMA]
    return pl.kernel(
        body,
        mesh=plsc.VectorSubcoreMesh(core_axis_name="c", subcore_axis_name="s"),
        out_type=out_type,
        scratch_types=scratch,
        compiler_params=pltpu.CompilerParams(use_tc_tiling_on_sc=False),
    )


_sc_layer1_p0 = _make_sc_layer1(0)
_sc_layer1_p1 = _make_sc_layer1(1)

EPT_B = E // NS
CH_B = 1000
NCH_B = EPT_B // CH_B


def _sc_layer2(p16, ei4, z16, out, srcb, dstb, rowsA, rowsB,
               abuf, pbuf, obuf, acc, semA, semB):
    c = lax.axis_index("c")
    s = lax.axis_index("s")

    @pl.when(c == 0)
    def _():
        pltpu.sync_copy(ei4.at[0, s], srcb)
        pltpu.sync_copy(ei4.at[1, s], dstb)
        pltpu.sync_copy(z16, acc.at[pl.ds(s * NPT, NPT)])
        plsc.subcore_barrier()

        bufs = (rowsA, rowsB)
        sems = (semA, semB)
        hg = {}
        for i in range(2):
            hg[i] = pltpu.async_copy(p16.at[srcb.at[i]], bufs[i], sems[i])
        for i in range(NCH_B):
            buf = bufs[i % 2]
            hg[i].wait()
            pltpu.sync_copy(buf, acc.at[dstb.at[i]], add=True)
            if i + 2 < NCH_B:
                hg[i + 2] = pltpu.async_copy(p16.at[srcb.at[i + 2]], buf,
                                             sems[i % 2])

        plsc.subcore_barrier()
        pltpu.sync_copy(acc.at[pl.ds(s * NPT, NPT)], abuf)
        pltpu.sync_copy(p16.at[pl.ds(s * NPT, NPT)], pbuf)

        def body(r, _):
            av = abuf[r]
            pv = pbuf[r]
            inv = pv[4]
            o0 = av[0] * inv + pv[NCLS]
            o1 = av[1] * inv + pv[NCLS + 1]
            lane = lax.iota(jnp.int32, 16)
            obuf[r] = jnp.where(lane == 0, o0, jnp.where(lane == 1, o1, 0.0))
            return 0

        lax.fori_loop(0, NPT, body, 0)
        pltpu.sync_copy(obuf, out.at[pl.ds(s * NPT, NPT)])


_sc_layer2_call = functools.partial(
    pl.kernel,
    mesh=plsc.VectorSubcoreMesh(core_axis_name="c", subcore_axis_name="s"),
    out_type=jax.ShapeDtypeStruct((NPAD, P16), jnp.float32),
    scratch_types=[
        pltpu.VMEM((NCH_B, CH_B), jnp.int32),
        pltpu.VMEM((NCH_B, CH_B), jnp.int32),
        pltpu.VMEM((CH_B, P16), jnp.float32),
        pltpu.VMEM((CH_B, P16), jnp.float32),
        pltpu.VMEM((NPT, P16), jnp.float32),
        pltpu.VMEM((NPT, P16), jnp.float32),
        pltpu.VMEM((NPT, P16), jnp.float32),
        pltpu.VMEM_SHARED((NPAD, P16), jnp.float32),
        pltpu.SemaphoreType.DMA,
        pltpu.SemaphoreType.DMA,
    ],
    compiler_params=pltpu.CompilerParams(use_tc_tiling_on_sc=False),
)(_sc_layer2)

BM = 2048


def _tc_root(x, b, b1, xb_out):
    xb_out[...] = (jnp.dot(x[...], b[...], preferred_element_type=jnp.float32)
                   + b1[...]).astype(jnp.bfloat16)


def _tc_root_call(x, b, b1):
    return pl.pallas_call(
        _tc_root,
        grid=(NPAD // BM,),
        in_specs=[
            pl.BlockSpec((BM, DIM), lambda i: (i, 0)),
            pl.BlockSpec((DIM, HID), lambda i: (0, 0)),
            pl.BlockSpec((1, HID), lambda i: (0, 0)),
        ],
        out_specs=pl.BlockSpec((BM, HID), lambda i: (i, 0)),
        out_shape=jax.ShapeDtypeStruct((NPAD, HID), jnp.bfloat16),
    )(x, b, b1)


def _tc_hidden(s0, s1, s2, s3, c0, c1, xb, a, w2cat, b2row, p16_out):
    cnt = c0[0][:, 0:1] + c1[0][:, 0:1]
    inv = 1.0 / jnp.maximum(cnt, 1.0)
    sfull = jnp.concatenate([s0[0], s1[0], s2[0], s3[0]], axis=1)
    zs = jnp.dot(sfull, a[...], preferred_element_type=jnp.float32)
    h = jnp.maximum(zs * inv + xb[...].astype(jnp.float32), 0.0)
    p = jnp.dot(h, w2cat[...], preferred_element_type=jnp.float32)
    col = lax.broadcasted_iota(jnp.int32, (BM, P16), 1)
    inv_col = jnp.where(col == 4, inv, 0.0)
    p16_out[...] = p + b2row[...] + inv_col


def _quarter_spec(q):
    return pl.BlockSpec((1, BM, FQ), lambda i, _q=q: (_q, i, 0))


def _tc_hidden_call(s01b, s23b, cnt, xb, a, w2cat, b2row):
    return pl.pallas_call(
        _tc_hidden,
        grid=(NPAD // BM,),
        in_specs=[
            _quarter_spec(0), _quarter_spec(1), _quarter_spec(0),
            _quarter_spec(1),
            pl.BlockSpec((1, BM, CB), lambda i: (0, i, 0)),
            pl.BlockSpec((1, BM, CB), lambda i: (1, i, 0)),
            pl.BlockSpec((BM, HID), lambda i: (i, 0)),
            pl.BlockSpec((DIM, HID), lambda i: (0, 0)),
            pl.BlockSpec((HID, P16), lambda i: (0, 0)),
            pl.BlockSpec((1, P16), lambda i: (0, 0)),
        ],
        out_specs=pl.BlockSpec((BM, P16), lambda i: (i, 0)),
        out_shape=jax.ShapeDtypeStruct((NPAD, P16), jnp.float32),
    )(s01b, s01b, s23b, s23b, cnt, cnt, xb, a, w2cat, b2row)


def kernel(x, edge_index, W1l, b1l, W1r, W2l, b2l, W2r):
    src = edge_index[0].astype(jnp.int32)
    dst = edge_index[1].astype(jnp.int32)

    xflat = x.reshape(4 * N, FQ)
    srcq = (src[None, :] * 4 + jnp.arange(4, dtype=jnp.int32)[:, None]).ravel()
    ei = jnp.stack([src, dst]) if edge_index.dtype != jnp.int32 else edge_index
    ei4a = ei.reshape(2, NS, NCH_A, CH_A)
    ones8 = jnp.ones((CH_A, CB), jnp.float32)
    z64 = jnp.zeros((NPT, FQ), jnp.float32)
    z8 = jnp.zeros((NPT, CB), jnp.float32)
    z16 = jnp.zeros((NPT, P16), jnp.float32)

    s01, cnt = _sc_layer1_p0(xflat, srcq, ei4a, ones8, z64, z8)
    s23, = _sc_layer1_p1(xflat, srcq, ei4a, ones8, z64, z8)
    xb = _tc_root_call(x.astype(jnp.bfloat16), W1r.T.astype(jnp.bfloat16),
                       b1l.reshape(1, HID))

    w2cat = jnp.concatenate(
        [W2l.T, W2r.T, jnp.zeros((HID, P16 - 2 * NCLS), jnp.float32)], axis=1)
    b2row = jnp.concatenate(
        [jnp.zeros((NCLS,), jnp.float32), b2l,
         jnp.zeros((P16 - 2 * NCLS,), jnp.float32)]).reshape(1, P16)
    p16 = _tc_hidden_call(s01.astype(jnp.bfloat16), s23.astype(jnp.bfloat16),
                          cnt, xb, W1l.T.astype(jnp.bfloat16), w2cat, b2row)

    ei4b = ei.reshape(2, NS, NCH_B, CH_B)
    out16 = _sc_layer2_call(p16, ei4b, z16)

    return out16[:N, :NCLS]


# speedup vs baseline: 1.0346x; 1.0042x over previous
import functools

import jax
import jax.numpy as jnp
from jax import lax
from jax.experimental import pallas as pl
from jax.experimental.pallas import tpu as pltpu
from jax.experimental.pallas import tpu_sc as plsc

N = 10000
E = 160000
DIM = 256
HID = 512
NCLS = 2

NC = 2
NS = 16
NPAD = 10240
FQ = 64
P16 = 16
NPT = NPAD // NS

EPT_A = E // NS
CH_A = 400
NCH_A = EPT_A // CH_A
NBUF = 2


CB = 8


def _make_sc_layer1(p):
    def body(xflat, srcq, ei4, ones8, z64, z8, *refs):
        if p == 0:
            out, cnt_out, srcgb, dstb, rows0, rows1, onesb, acc, cacc, \
                sem0, sem1 = refs
        else:
            out, srcgb, dstb, rows0, rows1, acc, sem0, sem1 = refs
        c = lax.axis_index("c")
        s = lax.axis_index("s")
        bufs = (rows0, rows1)
        sems = (sem0, sem1)

        pltpu.sync_copy(ei4.at[1, s], dstb)
        if p == 0:
            pltpu.sync_copy(ones8, onesb)
            pltpu.sync_copy(z8, cacc.at[pl.ds(s * NPT, NPT)])
        pltpu.sync_copy(z64, acc.at[pl.ds(s * NPT, NPT)])
        plsc.subcore_barrier()

        q = 2 * p + c
        pltpu.sync_copy(srcq.at[pl.ds(q * E + s * EPT_A, EPT_A)], srcgb)

        hg = {}
        for i in range(NBUF):
            hg[i] = pltpu.async_copy(
                xflat.at[srcgb.at[pl.ds(i * CH_A, CH_A)]], bufs[i], sems[i])
        for i in range(NCH_A):
            buf = bufs[i % NBUF]
            hg[i].wait()
            pltpu.sync_copy(buf, acc.at[dstb.at[i]], add=True)
            if p == 0:
                @pl.when(c == (i % 2))
                def _():
                    pltpu.sync_copy(onesb, cacc.at[dstb.at[i]], add=True)
            if i + NBUF < NCH_A:
                hg[i + NBUF] = pltpu.async_copy(
                    xflat.at[srcgb.at[pl.ds((i + NBUF) * CH_A, CH_A)]],
                    buf, sems[i % NBUF])

        plsc.subcore_barrier()
        pltpu.sync_copy(acc.at[pl.ds(s * NPT, NPT)],
                        out.at[c, pl.ds(s * NPT, NPT)])
        if p == 0:
            pltpu.sync_copy(cacc.at[pl.ds(s * NPT, NPT)],
                            cnt_out.at[c, pl.ds(s * NPT, NPT)])

    out_type = [jax.ShapeDtypeStruct((2, NPAD, FQ), jnp.float32)]
    scratch = [
        pltpu.VMEM((EPT_A,), jnp.int32),
        pltpu.VMEM((NCH_A, CH_A), jnp.int32),
        pltpu.VMEM((CH_A, FQ), jnp.float32),
        pltpu.VMEM((CH_A, FQ), jnp.float32),
    ]
    if p == 0:
        out_type.append(jax.ShapeDtypeStruct((2, NPAD, CB), jnp.float32))
        scratch.append(pltpu.VMEM((CH_A, CB), jnp.float32))
    scratch.append(pltpu.VMEM_SHARED((NPAD, FQ), jnp.float32))
    if p == 0:
        scratch.append(pltpu.VMEM_SHARED((NPAD, CB), jnp.float32))
    scratch += [pltpu.SemaphoreType.DMA, pltpu.SemaphoreType.DMA]
    return pl.kernel(
        body,
        mesh=plsc.VectorSubcoreMesh(core_axis_name="c", subcore_axis_name="s"),
        out_type=out_type,
        scratch_types=scratch,
        compiler_params=pltpu.CompilerParams(use_tc_tiling_on_sc=False),
    )


_sc_layer1_p0 = _make_sc_layer1(0)
_sc_layer1_p1 = _make_sc_layer1(1)

EPT_B = E // (NC * NS)
CH_B = 1000
NCH_B = EPT_B // CH_B
NPB = NPAD // (NC * NS)


def _sc_layer2(p16, ei4, z16, outp, srcb, dstb, rowsA, rowsB, acc,
               semA, semB):
    c = lax.axis_index("c")
    s = lax.axis_index("s")
    w = c * NS + s

    pltpu.sync_copy(ei4.at[0, w], srcb)
    pltpu.sync_copy(ei4.at[1, w], dstb)
    pltpu.sync_copy(z16, acc.at[pl.ds(s * NPT, NPT)])
    plsc.subcore_barrier()

    bufs = (rowsA, rowsB)
    sems = (semA, semB)
    hg = {}
    for i in range(2):
        hg[i] = pltpu.async_copy(p16.at[srcb.at[i]], bufs[i], sems[i])
    for i in range(NCH_B):
        buf = bufs[i % 2]
        hg[i].wait()
        pltpu.sync_copy(buf, acc.at[dstb.at[i]], add=True)
        if i + 2 < NCH_B:
            hg[i + 2] = pltpu.async_copy(p16.at[srcb.at[i + 2]], buf,
                                         sems[i % 2])

    plsc.subcore_barrier()
    pltpu.sync_copy(acc.at[pl.ds(s * NPT, NPT)],
                    outp.at[c, pl.ds(s * NPT, NPT)])


_sc_layer2_call = functools.partial(
    pl.kernel,
    mesh=plsc.VectorSubcoreMesh(core_axis_name="c", subcore_axis_name="s"),
    out_type=jax.ShapeDtypeStruct((2, NPAD, P16), jnp.float32),
    scratch_types=[
        pltpu.VMEM((NCH_B, CH_B), jnp.int32),
        pltpu.VMEM((NCH_B, CH_B), jnp.int32),
        pltpu.VMEM((CH_B, P16), jnp.float32),
        pltpu.VMEM((CH_B, P16), jnp.float32),
        pltpu.VMEM_SHARED((NPAD, P16), jnp.float32),
        pltpu.SemaphoreType.DMA,
        pltpu.SemaphoreType.DMA,
    ],
    compiler_params=pltpu.CompilerParams(use_tc_tiling_on_sc=False),
)(_sc_layer2)


def _sc_combine(outp, p16, out, a0buf, a1buf, pbuf, obuf):
    c = lax.axis_index("c")
    s = lax.axis_index("s")
    base = (c * NS + s) * NPB
    pltpu.sync_copy(outp.at[0, pl.ds(base, NPB)], a0buf)
    pltpu.sync_copy(outp.at[1, pl.ds(base, NPB)], a1buf)
    pltpu.sync_copy(p16.at[pl.ds(base, NPB)], pbuf)

    def body(r, _):
        av = a0buf[r] + a1buf[r]
        pv = pbuf[r]
        inv = pv[4]
        o0 = av[0] * inv + pv[NCLS]
        o1 = av[1] * inv + pv[NCLS + 1]
        lane = lax.iota(jnp.int32, 16)
        obuf[r] = jnp.where(lane == 0, o0, jnp.where(lane == 1, o1, 0.0))
        return 0

    lax.fori_loop(0, NPB, body, 0)
    pltpu.sync_copy(obuf, out.at[pl.ds(base, NPB)])


_sc_combine_call = functools.partial(
    pl.kernel,
    mesh=plsc.VectorSubcoreMesh(core_axis_name="c", subcore_axis_name="s"),
    out_type=jax.ShapeDtypeStruct((NPAD, P16), jnp.float32),
    scratch_types=[
        pltpu.VMEM((NPB, P16), jnp.float32),
        pltpu.VMEM((NPB, P16), jnp.float32),
        pltpu.VMEM((NPB, P16), jnp.float32),
        pltpu.VMEM((NPB, P16), jnp.float32),
    ],
    compiler_params=pltpu.CompilerParams(use_tc_tiling_on_sc=False),
)(_sc_combine)

BM = 2048


def _tc_root(x, b, b1, xb_out):
    xb_out[...] = (jnp.dot(x[...], b[...], preferred_element_type=jnp.float32)
                   + b1[...]).astype(jnp.bfloat16)


def _tc_root_call(x, b, b1):
    return pl.pallas_call(
        _tc_root,
        grid=(NPAD // BM,),
        in_specs=[
            pl.BlockSpec((BM, DIM), lambda i: (i, 0)),
            pl.BlockSpec((DIM, HID), lambda i: (0, 0)),
            pl.BlockSpec((1, HID), lambda i: (0, 0)),
        ],
        out_specs=pl.BlockSpec((BM, HID), lambda i: (i, 0)),
        out_shape=jax.ShapeDtypeStruct((NPAD, HID), jnp.bfloat16),
    )(x, b, b1)


def _tc_hidden(s0, s1, s2, s3, c0, c1, xb, a, w2cat, b2row, p16_out):
    cnt = c0[0][:, 0:1] + c1[0][:, 0:1]
    inv = 1.0 / jnp.maximum(cnt, 1.0)
    sfull = jnp.concatenate([s0[0], s1[0], s2[0], s3[0]], axis=1)
    zs = jnp.dot(sfull, a[...], preferred_element_type=jnp.float32)
    h = jnp.maximum(zs * inv + xb[...].astype(jnp.float32), 0.0)
    p = jnp.dot(h, w2cat[...], preferred_element_type=jnp.float32)
    col = lax.broadcasted_iota(jnp.int32, (BM, P16), 1)
    inv_col = jnp.where(col == 4, inv, 0.0)
    p16_out[...] = p + b2row[...] + inv_col


def _quarter_spec(q):
    return pl.BlockSpec((1, BM, FQ), lambda i, _q=q: (_q, i, 0))


def _tc_hidden_call(s01b, s23b, cnt, xb, a, w2cat, b2row):
    return pl.pallas_call(
        _tc_hidden,
        grid=(NPAD // BM,),
        in_specs=[
            _quarter_spec(0), _quarter_spec(1), _quarter_spec(0),
            _quarter_spec(1),
            pl.BlockSpec((1, BM, CB), lambda i: (0, i, 0)),
            pl.BlockSpec((1, BM, CB), lambda i: (1, i, 0)),
            pl.BlockSpec((BM, HID), lambda i: (i, 0)),
            pl.BlockSpec((DIM, HID), lambda i: (0, 0)),
            pl.BlockSpec((HID, P16), lambda i: (0, 0)),
            pl.BlockSpec((1, P16), lambda i: (0, 0)),
        ],
        out_specs=pl.BlockSpec((BM, P16), lambda i: (i, 0)),
        out_shape=jax.ShapeDtypeStruct((NPAD, P16), jnp.float32),
    )(s01b, s01b, s23b, s23b, cnt, cnt, xb, a, w2cat, b2row)


def kernel(x, edge_index, W1l, b1l, W1r, W2l, b2l, W2r):
    src = edge_index[0].astype(jnp.int32)
    dst = edge_index[1].astype(jnp.int32)

    xflat = x.reshape(4 * N, FQ)
    srcq = (src[None, :] * 4 + jnp.arange(4, dtype=jnp.int32)[:, None]).ravel()
    ei = jnp.stack([src, dst]) if edge_index.dtype != jnp.int32 else edge_index
    ei4a = ei.reshape(2, NS, NCH_A, CH_A)
    ones8 = jnp.ones((CH_A, CB), jnp.float32)
    z64 = jnp.zeros((NPT, FQ), jnp.float32)
    z8 = jnp.zeros((NPT, CB), jnp.float32)
    z16 = jnp.zeros((NPT, P16), jnp.float32)

    s01, cnt = _sc_layer1_p0(xflat, srcq, ei4a, ones8, z64, z8)
    s23, = _sc_layer1_p1(xflat, srcq, ei4a, ones8, z64, z8)
    xb = _tc_root_call(x.astype(jnp.bfloat16), W1r.T.astype(jnp.bfloat16),
                       b1l.reshape(1, HID))

    w2cat = jnp.concatenate(
        [W2l.T, W2r.T, jnp.zeros((HID, P16 - 2 * NCLS), jnp.float32)], axis=1)
    b2row = jnp.concatenate(
        [jnp.zeros((NCLS,), jnp.float32), b2l,
         jnp.zeros((P16 - 2 * NCLS,), jnp.float32)]).reshape(1, P16)
    p16 = _tc_hidden_call(s01.astype(jnp.bfloat16), s23.astype(jnp.bfloat16),
                          cnt, xb, W1l.T.astype(jnp.bfloat16), w2cat, b2row)

    ei4b = ei.reshape(2, NC * NS, NCH_B, CH_B)
    agg2 = _sc_layer2_call(p16, ei4b, z16)
    out16 = _sc_combine_call(agg2, p16)

    return out16[:N, :NCLS]
